# baseline (speedup 1.0000x reference)
import jax
import jax.numpy as jnp
from jax import lax
from jax.experimental import pallas as pl
from jax.experimental.pallas import tpu as pltpu
from jax.experimental.pallas import tpu_sc as plsc

N_NODES = 10000
N_EDGES = 320000
D_IN = 128
D_HID = 256
D_OUT = 128

NC = 2
NS = 16
NW = NC * NS
CHUNK = 64
NGRP = 8
GCH = 20
NCHUNK = NGRP * GCH
EPT = NCHUNK * CHUNK
E_PAD = NW * EPT - N_EDGES
NBUF = 4
RPT = 624
TAIL = N_NODES - NS * RPT
TAIL_OFF = NS * RPT

_sc_mesh = plsc.VectorSubcoreMesh(
    core_axis_name="c", subcore_axis_name="s", num_cores=NC, num_subcores=NS
)


def _sc_agg_body(x_hbm, edges_hbm, out_hbm, src_v, dst_v, rows_v, sems,
                 agg_sh):
    c = lax.axis_index("c")
    s = lax.axis_index("s")
    wid = c * NS + s

    pltpu.sync_copy(x_hbm.at[pl.ds(s * RPT, RPT)],
                    agg_sh.at[pl.ds(s * RPT, RPT)])

    @pl.when(s == 0)
    def _seed_tail():
        pltpu.sync_copy(x_hbm.at[pl.ds(TAIL_OFF, TAIL)],
                        agg_sh.at[pl.ds(TAIL_OFF, TAIL)])

    plsc.subcore_barrier()

    n_full = (GCH - NBUF - 1) // NBUF
    for g in range(NGRP):
        pltpu.sync_copy(edges_hbm.at[0, wid, g], src_v)
        pltpu.sync_copy(edges_hbm.at[1, wid, g], dst_v)
        for b in range(NBUF):
            pltpu.async_copy(x_hbm.at[src_v.at[b]], rows_v.at[b], sems[b])

        def ring(i, carry):
            j = NBUF * i
            for b in range(NBUF):
                pltpu.make_async_copy(x_hbm.at[src_v.at[j + b]],
                                      rows_v.at[b], sems[b]).wait()
                pltpu.sync_copy(rows_v.at[b], agg_sh.at[dst_v.at[j + b]],
                                add=True)
                pltpu.async_copy(x_hbm.at[src_v.at[j + NBUF + b]],
                                 rows_v.at[b], sems[b])
            return carry

        lax.fori_loop(0, n_full, ring, 0)
        for l in range(NBUF * n_full, GCH):
            b = l % NBUF
            pltpu.make_async_copy(x_hbm.at[src_v.at[l]],
                                  rows_v.at[b], sems[b]).wait()
            pltpu.sync_copy(rows_v.at[b], agg_sh.at[dst_v.at[l]], add=True)
            if l + NBUF < GCH:
                pltpu.async_copy(x_hbm.at[src_v.at[l + NBUF]],
                                 rows_v.at[b], sems[b])
    plsc.subcore_barrier()
    pltpu.sync_copy(edges_hbm.at[0, wid, 0], src_v)
    plsc.subcore_barrier()

    pltpu.sync_copy(agg_sh.at[pl.ds(s * RPT, RPT)],
                    out_hbm.at[c, pl.ds(s * RPT, RPT)])

    @pl.when(s == 0)
    def _write_tail():
        pltpu.sync_copy(agg_sh.at[pl.ds(TAIL_OFF, TAIL)],
                        out_hbm.at[c, pl.ds(TAIL_OFF, TAIL)])


_sc_agg = pl.kernel(
    _sc_agg_body,
    out_type=jax.ShapeDtypeStruct((NC, N_NODES, D_IN), jnp.float32),
    mesh=_sc_mesh,
    scratch_types=[
        pltpu.VMEM((GCH, CHUNK), jnp.int32),
        pltpu.VMEM((GCH, CHUNK), jnp.int32),
        pltpu.VMEM((NBUF, CHUNK, D_IN), jnp.float32),
        [pltpu.SemaphoreType.DMA] * NBUF,
        pltpu.VMEM_SHARED((N_NODES + 16, D_IN), jnp.float32),
    ],
)


def _mlp_body(eps_ref, x_ref, p_ref, w1_ref, b1_ref, w2_ref, b2_ref, o_ref):
    z = x_ref[...] * (eps_ref[0, 0] - 1.0) + p_ref[0] + p_ref[1]
    h = jnp.dot(z, w1_ref[...], preferred_element_type=jnp.float32)
    h = jnp.maximum(h + b1_ref[...], 0.0)
    o = jnp.dot(h, w2_ref[...], preferred_element_type=jnp.float32)
    o_ref[...] = o + b2_ref[...]


_ROWS_BLK = 2000


def _mlp(eps2d, x, partials, W1, b1, W2, b2):
    grid = (N_NODES // _ROWS_BLK,)
    return pl.pallas_call(
        _mlp_body,
        grid=grid,
        in_specs=[
            pl.BlockSpec(memory_space=pltpu.SMEM),
            pl.BlockSpec((_ROWS_BLK, D_IN), lambda i: (i, 0)),
            pl.BlockSpec((NC, _ROWS_BLK, D_IN), lambda i: (0, i, 0)),
            pl.BlockSpec((D_IN, D_HID), lambda i: (0, 0)),
            pl.BlockSpec((1, D_HID), lambda i: (0, 0)),
            pl.BlockSpec((D_HID, D_OUT), lambda i: (0, 0)),
            pl.BlockSpec((1, D_OUT), lambda i: (0, 0)),
        ],
        out_specs=pl.BlockSpec((_ROWS_BLK, D_OUT), lambda i: (i, 0)),
        out_shape=jax.ShapeDtypeStruct((N_NODES, D_OUT), jnp.float32),
    )(eps2d, x, partials, W1, b1, W2, b2)


def kernel(x, edge_index, eps, W1, b1, W2, b2):
    pad = jnp.stack([jnp.zeros((E_PAD,), jnp.int32),
                     jnp.full((E_PAD,), N_NODES, jnp.int32)])
    edges = jnp.concatenate([edge_index, pad], axis=1)
    edges = edges.reshape(2, NW, NGRP, GCH, CHUNK)
    partials = _sc_agg(x, edges)
    eps2d = eps.reshape(1, 1).astype(jnp.float32)
    return _mlp(eps2d, x, partials,
                W1, b1.reshape(1, D_HID), W2, b2.reshape(1, D_OUT))


# speedup vs baseline: 3.6688x; 3.6688x over previous
import jax
import jax.numpy as jnp
from jax import lax
from jax.experimental import pallas as pl
from jax.experimental.pallas import tpu as pltpu
from jax.experimental.pallas import tpu_sc as plsc

N_NODES = 10000
N_EDGES = 320000
D_IN = 128
D_HID = 256
D_OUT = 128

NC = 2
NS = 16
NW = NC * NS
EPT = N_EDGES // NW
CHUNK = 80
NCHUNK = EPT // CHUNK
NGRP = 5
GCH = NCHUNK // NGRP
NBUF = 3
RPT = 624
TAIL = N_NODES - NS * RPT
TAIL_OFF = NS * RPT

_sc_mesh = plsc.VectorSubcoreMesh(
    core_axis_name="c", subcore_axis_name="s", num_cores=NC, num_subcores=NS
)


def _sc_agg_body(x_hbm, edges_hbm, out_hbm, src_v, dst_v, rows_v, sems,
                 agg_sh):
    c = lax.axis_index("c")
    s = lax.axis_index("s")
    wid = c * NS + s

    pltpu.sync_copy(x_hbm.at[pl.ds(s * RPT, RPT)],
                    agg_sh.at[pl.ds(s * RPT, RPT)])

    @pl.when(s == 0)
    def _seed_tail():
        pltpu.sync_copy(x_hbm.at[pl.ds(TAIL_OFF, TAIL)],
                        agg_sh.at[pl.ds(TAIL_OFF, TAIL)])

    plsc.subcore_barrier()

    n_full = (GCH - NBUF - 1) // NBUF
    for g in range(NGRP):
        pltpu.sync_copy(edges_hbm.at[0, wid, g], src_v)
        pltpu.sync_copy(edges_hbm.at[1, wid, g], dst_v)
        for b in range(NBUF):
            pltpu.async_copy(x_hbm.at[src_v.at[b]], rows_v.at[b], sems[b])

        def ring(i, carry):
            j = NBUF * i
            for b in range(NBUF):
                pltpu.make_async_copy(x_hbm.at[src_v.at[j + b]],
                                      rows_v.at[b], sems[b]).wait()
                pltpu.sync_copy(rows_v.at[b], agg_sh.at[dst_v.at[j + b]],
                                add=True)
                pltpu.async_copy(x_hbm.at[src_v.at[j + NBUF + b]],
                                 rows_v.at[b], sems[b])
            return carry

        lax.fori_loop(0, n_full, ring, 0)
        for l in range(NBUF * n_full, GCH):
            b = l % NBUF
            pltpu.make_async_copy(x_hbm.at[src_v.at[l]],
                                  rows_v.at[b], sems[b]).wait()
            pltpu.sync_copy(rows_v.at[b], agg_sh.at[dst_v.at[l]], add=True)
            if l + NBUF < GCH:
                pltpu.async_copy(x_hbm.at[src_v.at[l + NBUF]],
                                 rows_v.at[b], sems[b])
    plsc.subcore_barrier()
    pltpu.sync_copy(edges_hbm.at[0, wid, 0], src_v)
    plsc.subcore_barrier()

    pltpu.sync_copy(agg_sh.at[pl.ds(s * RPT, RPT)],
                    out_hbm.at[c, pl.ds(s * RPT, RPT)])

    @pl.when(s == 0)
    def _write_tail():
        pltpu.sync_copy(agg_sh.at[pl.ds(TAIL_OFF, TAIL)],
                        out_hbm.at[c, pl.ds(TAIL_OFF, TAIL)])


_sc_agg = pl.kernel(
    _sc_agg_body,
    out_type=jax.ShapeDtypeStruct((NC, N_NODES, D_IN), jnp.float32),
    mesh=_sc_mesh,
    scratch_types=[
        pltpu.VMEM((GCH, CHUNK), jnp.int32),
        pltpu.VMEM((GCH, CHUNK), jnp.int32),
        pltpu.VMEM((NBUF, CHUNK, D_IN), jnp.float32),
        [pltpu.SemaphoreType.DMA] * NBUF,
        pltpu.VMEM_SHARED((N_NODES, D_IN), jnp.float32),
    ],
)


def _mlp_body(eps_ref, x_ref, p_ref, w1_ref, b1_ref, w2_ref, b2_ref, o_ref):
    z = x_ref[...] * (eps_ref[0, 0] - 1.0) + p_ref[0] + p_ref[1]
    h = jnp.dot(z, w1_ref[...], preferred_element_type=jnp.float32)
    h = jnp.maximum(h + b1_ref[...], 0.0)
    o = jnp.dot(h, w2_ref[...], preferred_element_type=jnp.float32)
    o_ref[...] = o + b2_ref[...]


_ROWS_BLK = 2000


def _mlp(eps2d, x, partials, W1, b1, W2, b2):
    grid = (N_NODES // _ROWS_BLK,)
    return pl.pallas_call(
        _mlp_body,
        grid=grid,
        in_specs=[
            pl.BlockSpec(memory_space=pltpu.SMEM),
            pl.BlockSpec((_ROWS_BLK, D_IN), lambda i: (i, 0)),
            pl.BlockSpec((NC, _ROWS_BLK, D_IN), lambda i: (0, i, 0)),
            pl.BlockSpec((D_IN, D_HID), lambda i: (0, 0)),
            pl.BlockSpec((1, D_HID), lambda i: (0, 0)),
            pl.BlockSpec((D_HID, D_OUT), lambda i: (0, 0)),
            pl.BlockSpec((1, D_OUT), lambda i: (0, 0)),
        ],
        out_specs=pl.BlockSpec((_ROWS_BLK, D_OUT), lambda i: (i, 0)),
        out_shape=jax.ShapeDtypeStruct((N_NODES, D_OUT), jnp.float32),
    )(eps2d, x, partials, W1, b1, W2, b2)


def kernel(x, edge_index, eps, W1, b1, W2, b2):
    edges = edge_index.reshape(2, NW, NGRP, GCH, CHUNK)
    partials = _sc_agg(x, edges)
    eps2d = eps.reshape(1, 1).astype(jnp.float32)
    return _mlp(eps2d, x, partials,
                W1, b1.reshape(1, D_HID), W2, b2.reshape(1, D_OUT))


# speedup vs baseline: 3.9412x; 1.0743x over previous
import jax
import jax.numpy as jnp
from jax import lax
from jax.experimental import pallas as pl
from jax.experimental.pallas import tpu as pltpu
from jax.experimental.pallas import tpu_sc as plsc

N_NODES = 10000
N_EDGES = 320000
D_IN = 128
D_HID = 256
D_OUT = 128

NC = 2
NS = 16
NW = NC * NS
EPT = N_EDGES // NW
CHUNK = 80
NCHUNK = EPT // CHUNK
NGRP = 5
GCH = NCHUNK // NGRP
NBUF = 3
RPT = 624
TAIL = N_NODES - NS * RPT
TAIL_OFF = NS * RPT

_sc_mesh = plsc.VectorSubcoreMesh(
    core_axis_name="c", subcore_axis_name="s", num_cores=NC, num_subcores=NS
)


def _sc_agg_body(x_hbm, edges_hbm, out_hbm, src_v, dst_v, rows_v, sems,
                 sidx, agg_sh):
    c = lax.axis_index("c")
    s = lax.axis_index("s")
    wid = c * NS + s

    pltpu.sync_copy(x_hbm.at[pl.ds(s * RPT, RPT)],
                    agg_sh.at[pl.ds(s * RPT, RPT)])

    @pl.when(s == 0)
    def _seed_tail():
        pltpu.sync_copy(x_hbm.at[pl.ds(TAIL_OFF, TAIL)],
                        agg_sh.at[pl.ds(TAIL_OFF, TAIL)])

    plsc.subcore_barrier()

    pltpu.sync_copy(edges_hbm.at[0, wid, 0], src_v.at[0])
    pltpu.sync_copy(edges_hbm.at[1, wid, 0], dst_v)
    for b in range(NBUF):
        pltpu.async_copy(x_hbm.at[src_v.at[0, b]], rows_v.at[b], sems[b])

    for g in range(NGRP):
        p = g % 2
        q = (g + 1) % 2
        phi = g % NBUF
        if g > 0:
            pltpu.sync_copy(edges_hbm.at[1, wid, g], dst_v)
        if g + 1 < NGRP:
            pltpu.async_copy(edges_hbm.at[0, wid, g + 1], src_v.at[q], sidx)

        def ring(i, carry, p=p, phi=phi):
            j = NBUF * i
            for b in range(NBUF):
                bb = (phi + b) % NBUF
                pltpu.make_async_copy(x_hbm.at[src_v.at[p, j + b]],
                                      rows_v.at[bb], sems[bb]).wait()
                pltpu.sync_copy(rows_v.at[bb], agg_sh.at[dst_v.at[j + b]],
                                add=True)
                pltpu.async_copy(x_hbm.at[src_v.at[p, j + NBUF + b]],
                                 rows_v.at[bb], sems[bb])
            return carry

        n_full = (GCH - NBUF - 1) // NBUF
        lax.fori_loop(0, n_full, ring, 0)
        for jj in range(NBUF * n_full, GCH - NBUF):
            bb = (phi + jj) % NBUF
            pltpu.make_async_copy(x_hbm.at[src_v.at[p, jj]],
                                  rows_v.at[bb], sems[bb]).wait()
            pltpu.sync_copy(rows_v.at[bb], agg_sh.at[dst_v.at[jj]], add=True)
            pltpu.async_copy(x_hbm.at[src_v.at[p, jj + NBUF]],
                             rows_v.at[bb], sems[bb])
        if g + 1 < NGRP:
            pltpu.make_async_copy(edges_hbm.at[0, wid, g + 1], src_v.at[q],
                                  sidx).wait()
        for t in range(NBUF):
            jj = GCH - NBUF + t
            bb = (phi + jj) % NBUF
            pltpu.make_async_copy(x_hbm.at[src_v.at[p, jj]],
                                  rows_v.at[bb], sems[bb]).wait()
            pltpu.sync_copy(rows_v.at[bb], agg_sh.at[dst_v.at[jj]], add=True)
            if g + 1 < NGRP:
                pltpu.async_copy(x_hbm.at[src_v.at[q, t]], rows_v.at[bb],
                                 sems[bb])
    plsc.subcore_barrier()
    pltpu.sync_copy(edges_hbm.at[0, wid, 0], src_v.at[0])
    plsc.subcore_barrier()

    pltpu.sync_copy(agg_sh.at[pl.ds(s * RPT, RPT)],
                    out_hbm.at[c, pl.ds(s * RPT, RPT)])

    @pl.when(s == 0)
    def _write_tail():
        pltpu.sync_copy(agg_sh.at[pl.ds(TAIL_OFF, TAIL)],
                        out_hbm.at[c, pl.ds(TAIL_OFF, TAIL)])


_sc_agg = pl.kernel(
    _sc_agg_body,
    out_type=jax.ShapeDtypeStruct((NC, N_NODES, D_IN), jnp.float32),
    mesh=_sc_mesh,
    scratch_types=[
        pltpu.VMEM((2, GCH, CHUNK), jnp.int32),
        pltpu.VMEM((GCH, CHUNK), jnp.int32),
        pltpu.VMEM((NBUF, CHUNK, D_IN), jnp.float32),
        [pltpu.SemaphoreType.DMA] * NBUF,
        pltpu.SemaphoreType.DMA,
        pltpu.VMEM_SHARED((N_NODES, D_IN), jnp.float32),
    ],
)


def _mlp_body(eps_ref, x_ref, p_ref, w1_ref, b1_ref, w2_ref, b2_ref, o_ref):
    z = x_ref[...] * (eps_ref[0, 0] - 1.0) + p_ref[0] + p_ref[1]
    h = jnp.dot(z, w1_ref[...], preferred_element_type=jnp.float32)
    h = jnp.maximum(h + b1_ref[...], 0.0)
    o = jnp.dot(h, w2_ref[...], preferred_element_type=jnp.float32)
    o_ref[...] = o + b2_ref[...]


_ROWS_BLK = 2000


def _mlp(eps2d, x, partials, W1, b1, W2, b2):
    grid = (N_NODES // _ROWS_BLK,)
    return pl.pallas_call(
        _mlp_body,
        grid=grid,
        in_specs=[
            pl.BlockSpec(memory_space=pltpu.SMEM),
            pl.BlockSpec((_ROWS_BLK, D_IN), lambda i: (i, 0)),
            pl.BlockSpec((NC, _ROWS_BLK, D_IN), lambda i: (0, i, 0)),
            pl.BlockSpec((D_IN, D_HID), lambda i: (0, 0)),
            pl.BlockSpec((1, D_HID), lambda i: (0, 0)),
            pl.BlockSpec((D_HID, D_OUT), lambda i: (0, 0)),
            pl.BlockSpec((1, D_OUT), lambda i: (0, 0)),
        ],
        out_specs=pl.BlockSpec((_ROWS_BLK, D_OUT), lambda i: (i, 0)),
        out_shape=jax.ShapeDtypeStruct((N_NODES, D_OUT), jnp.float32),
    )(eps2d, x, partials, W1, b1, W2, b2)


def kernel(x, edge_index, eps, W1, b1, W2, b2):
    edges = edge_index.reshape(2, NW, NGRP, GCH, CHUNK)
    partials = _sc_agg(x, edges)
    eps2d = eps.reshape(1, 1).astype(jnp.float32)
    return _mlp(eps2d, x, partials,
                W1, b1.reshape(1, D_HID), W2, b2.reshape(1, D_OUT))
